# Initial kernel scaffold; baseline (speedup 1.0000x reference)
import jax
import jax.numpy as jnp
from jax.experimental import pallas as pl


def kernel(r, phi_r, trainables_m, trainables_p):
    raise NotImplementedError("write your pallas kernel here")


# speedup vs baseline: 30.3640x; 30.3640x over previous
import functools

import jax
import jax.numpy as jnp
from jax import lax
from jax.experimental import pallas as pl
from jax.experimental.pallas import tpu as pltpu
from jax.experimental.pallas import tpu_sc as plsc

NX = NY = NZ = 256
NXYZ = NX * NY * NZ
N_POINTS = 524288
NW = 32
PTS_PER_W = N_POINTS // NW
C = 128
G = C // 16
NCHUNK = PTS_PER_W // C
NSLOT = 32

_SLOTS = (
    [(c, d, e) for (c, d) in ((1, 1), (1, 2), (2, 1), (2, 2)) for e in range(4)]
    + [(c, d, e) for c in (0, 3) for d in (1, 2) for e in (1, 2)]
    + [(c, d, e) for c in (1, 2) for d in (0, 3) for e in (1, 2)]
)
assert len(_SLOTS) == NSLOT


def _axis_calc(p, coord_v, xg0, dxv):
    t = (p - xg0) / dxv
    i = t.astype(jnp.int32)
    i = jnp.clip(i, 2, 256)
    ci = plsc.load_gather(coord_v, [i])
    ci1 = plsc.load_gather(coord_v, [i + 1])
    fd = (p - ci) / (ci1 - ci)
    a = i - 1
    x0 = a - 1
    x2 = jnp.minimum(a + 1, NX - 1)
    x3 = jnp.minimum(a + 2, NX - 1)
    return fd, (x0, a, x2, x3)


def _make_sc_call():
    mesh = plsc.VectorSubcoreMesh(core_axis_name="c", subcore_axis_name="s")

    @functools.partial(
        pl.kernel,
        out_type=jax.ShapeDtypeStruct((N_POINTS,), jnp.float32),
        mesh=mesh,
        compiler_params=pltpu.CompilerParams(needs_layout_passes=False),
        scratch_types=[
            pltpu.VMEM((384,), jnp.float32),
            pltpu.VMEM((32,), jnp.float32),
            pltpu.VMEM((C,), jnp.float32),
            pltpu.VMEM((C,), jnp.float32),
            pltpu.VMEM((C,), jnp.float32),
            pltpu.VMEM((C,), jnp.float32),
            pltpu.VMEM((NSLOT, C), jnp.int32),
            pltpu.VMEM((NSLOT, C), jnp.float32),
            pltpu.VMEM((C,), jnp.float32),
            pltpu.SemaphoreType.DMA,
        ],
    )
    def body(xs_hbm, ys_hbm, zs_hbm, phi_hbm, tb_hbm, xg_hbm, cst_hbm, out_hbm,
             coord_v, cst_v, xv, yv, zv, pv, idx_v, val_v, res_v, sem):
        wid = lax.axis_index("s") * 2 + lax.axis_index("c")
        pltpu.sync_copy(xg_hbm, coord_v)
        pltpu.sync_copy(cst_hbm, cst_v)
        xg0 = cst_v[pl.ds(0, 16)]
        dxv = cst_v[pl.ds(16, 16)]

        def chunk(t, carry):
            base = wid * PTS_PER_W + t * C
            pltpu.sync_copy(xs_hbm.at[pl.ds(base, C)], xv)
            pltpu.sync_copy(ys_hbm.at[pl.ds(base, C)], yv)
            pltpu.sync_copy(zs_hbm.at[pl.ds(base, C)], zv)
            pltpu.sync_copy(phi_hbm.at[pl.ds(base, C)], pv)

            for g in range(G):
                sl = pl.ds(g * 16, 16)
                fx, xi = _axis_calc(xv[sl], coord_v, xg0, dxv)
                fy, yi = _axis_calc(yv[sl], coord_v, xg0, dxv)
                fz, zi = _axis_calc(zv[sl], coord_v, xg0, dxv)
                off = jnp.where(pv[sl] >= 0.0,
                                jnp.int32(NXYZ), jnp.int32(0))
                cols = {}
                for (c, d, _e) in _SLOTS:
                    if (c, d) not in cols:
                        cols[(c, d)] = (xi[c] * NY + yi[d]) * NZ + off
                for s, (c, d, e) in enumerate(_SLOTS):
                    idx_v[s, pl.ds(g * 16, 16)] = cols[(c, d)] + zi[e]

            copies = [
                pltpu.async_copy(tb_hbm.at[idx_v.at[s]],
                                 val_v.at[s], sem)
                for s in range(NSLOT)
            ]
            for cp in copies:
                cp.wait()

            for g in range(G):
                sl = pl.ds(g * 16, 16)
                fx, _ = _axis_calc(xv[sl], coord_v, xg0, dxv)
                fy, _ = _axis_calc(yv[sl], coord_v, xg0, dxv)
                fz, _ = _axis_calc(zv[sl], coord_v, xg0, dxv)
                v = {
                    cde: val_v[s, pl.ds(g * 16, 16)]
                    for s, cde in enumerate(_SLOTS)
                }
                c00 = v[1, 1, 1] * (1.0 - fx) + v[2, 1, 1] * fx
                c01 = v[1, 1, 2] * (1.0 - fx) + v[2, 1, 2] * fx
                c10 = v[1, 2, 1] * (1.0 - fx) + v[2, 2, 1] * fx
                c11 = v[1, 2, 2] * (1.0 - fx) + v[2, 2, 2] * fx
                c0 = c00 * (1.0 - fy) + c10 * fy
                c1 = c01 * (1.0 - fy) + c11 * fy
                cval = c0 * (1.0 - fz) + c1 * fz
                mdx = mdy = mdz = None
                for (c, d, e) in ((1, 1, 1), (2, 1, 1), (1, 2, 1), (1, 1, 2),
                                  (2, 1, 2), (1, 2, 2), (2, 2, 1), (2, 2, 2)):
                    d2x = jnp.abs(v[c + 1, d, e] - 2.0 * v[c, d, e]
                                  + v[c - 1, d, e])
                    d2y = jnp.abs(v[c, d + 1, e] - 2.0 * v[c, d, e]
                                  + v[c, d - 1, e])
                    d2z = jnp.abs(v[c, d, e + 1] - 2.0 * v[c, d, e]
                                  + v[c, d, e - 1])
                    mdx = d2x if mdx is None else jnp.minimum(mdx, d2x)
                    mdy = d2y if mdy is None else jnp.minimum(mdy, d2y)
                    mdz = d2z if mdz is None else jnp.minimum(mdz, d2z)
                cval = (cval
                        - mdx * 0.5 * fx * (1.0 - fx)
                        - mdy * 0.5 * fy * (1.0 - fy)
                        - mdz * 0.5 * fz * (1.0 - fz))
                res_v[sl] = cval

            pltpu.sync_copy(res_v, out_hbm.at[pl.ds(base, C)])
            return carry

        lax.fori_loop(0, NCHUNK, chunk, 0)

    return body


_SC_CALL = _make_sc_call()


def kernel(r, phi_r, trainables_m, trainables_p):
    xc = jnp.linspace(-1.0, 1.0, NX, dtype=jnp.float32)
    dx = xc[1] - xc[0]
    xg = jnp.concatenate([xc[:1] - dx, xc, xc[-1:] + dx])
    xg = jnp.pad(xg, (0, 384 - NX - 2))
    tb = jnp.concatenate([trainables_m.reshape(-1), trainables_p.reshape(-1)])
    cst = jnp.concatenate([jnp.full((16,), xg[0]), jnp.full((16,), dx)])
    rt = r.T
    return _SC_CALL(rt[0], rt[1], rt[2], phi_r, tb, xg, cst)


# speedup vs baseline: 44.5912x; 1.4686x over previous
import functools

import jax
import jax.numpy as jnp
from jax import lax
from jax.experimental import pallas as pl
from jax.experimental.pallas import tpu as pltpu
from jax.experimental.pallas import tpu_sc as plsc

NX = NY = NZ = 256
NXYZ = NX * NY * NZ
N_POINTS = 524288
NW = 32
PTS_PER_W = N_POINTS // NW
C = 128
G = C // 16
NCHUNK = PTS_PER_W // C
NSLOT = 32

_SLOTS = (
    [(c, d, e) for (c, d) in ((1, 1), (1, 2), (2, 1), (2, 2)) for e in range(4)]
    + [(c, d, e) for c in (0, 3) for d in (1, 2) for e in (1, 2)]
    + [(c, d, e) for c in (1, 2) for d in (0, 3) for e in (1, 2)]
)
assert len(_SLOTS) == NSLOT
_D2_CORNERS = ((1, 1, 1), (2, 1, 1), (1, 2, 1), (1, 1, 2),
               (2, 1, 2), (1, 2, 2), (2, 2, 1), (2, 2, 2))


def _make_sc_call():
    mesh = plsc.VectorSubcoreMesh(core_axis_name="c", subcore_axis_name="s")

    @functools.partial(
        pl.kernel,
        out_type=jax.ShapeDtypeStruct((N_POINTS,), jnp.float32),
        mesh=mesh,
        compiler_params=pltpu.CompilerParams(needs_layout_passes=False),
        scratch_types=[
            pltpu.VMEM((384,), jnp.float32),
            pltpu.VMEM((32,), jnp.float32),
            pltpu.VMEM((PTS_PER_W,), jnp.float32),
            pltpu.VMEM((PTS_PER_W,), jnp.float32),
            pltpu.VMEM((PTS_PER_W,), jnp.float32),
            pltpu.VMEM((PTS_PER_W,), jnp.float32),
            pltpu.VMEM((PTS_PER_W,), jnp.float32),
            pltpu.VMEM((NSLOT, C), jnp.int32),
            pltpu.VMEM((NSLOT, C), jnp.int32),
            pltpu.VMEM((NSLOT, C), jnp.float32),
            pltpu.VMEM((NSLOT, C), jnp.float32),
            pltpu.VMEM((4, C), jnp.float32),
            pltpu.VMEM((4, C), jnp.float32),
            pltpu.SemaphoreType.DMA,
            pltpu.SemaphoreType.DMA,
        ],
    )
    def body(xs_hbm, ys_hbm, zs_hbm, phi_hbm, tb_hbm, xg_hbm, cst_hbm,
             out_hbm, coord_v, cst_v, xv, yv, zv, pv, rv,
             idx_a, idx_b, val_a, val_b, frac_a, frac_b, sem_a, sem_b):
        wid = lax.axis_index("s") * 2 + lax.axis_index("c")
        tbase = wid * PTS_PER_W
        pltpu.sync_copy(xg_hbm, coord_v)
        pltpu.sync_copy(cst_hbm, cst_v)
        pltpu.sync_copy(xs_hbm.at[pl.ds(tbase, PTS_PER_W)], xv)
        pltpu.sync_copy(ys_hbm.at[pl.ds(tbase, PTS_PER_W)], yv)
        pltpu.sync_copy(zs_hbm.at[pl.ds(tbase, PTS_PER_W)], zv)
        pltpu.sync_copy(phi_hbm.at[pl.ds(tbase, PTS_PER_W)], pv)
        xg0 = cst_v[pl.ds(0, 16)]
        dxv = cst_v[pl.ds(16, 16)]

        def axis_calc(p):
            t = (p - xg0) / dxv
            i = t.astype(jnp.int32)
            i = jnp.clip(i, 2, 256)
            ci = plsc.load_gather(coord_v, [i])
            ci1 = plsc.load_gather(coord_v, [i + 1])
            fd = (p - ci) / (ci1 - ci)
            a = i - 1
            return fd, (a - 1, a, jnp.minimum(a + 1, NX - 1),
                        jnp.minimum(a + 2, NX - 1))

        def fire(t, idx_v, frac_v, sem):
            def grp(g, carry):
                sl = pl.ds(g * 16, 16)
                psl = pl.ds(t * C + g * 16, 16)
                fx, xi = axis_calc(xv[psl])
                fy, yi = axis_calc(yv[psl])
                fz, zi = axis_calc(zv[psl])
                off = jnp.where(pv[psl] >= 0.0, jnp.int32(NXYZ), jnp.int32(0))
                frac_v[0, sl] = fx
                frac_v[1, sl] = fy
                frac_v[2, sl] = fz
                cols = {}
                for (c, d, _e) in _SLOTS:
                    if (c, d) not in cols:
                        cols[(c, d)] = (xi[c] * NY + yi[d]) * NZ + off
                for s, (c, d, e) in enumerate(_SLOTS):
                    idx_v[s, sl] = cols[(c, d)] + zi[e]
                return carry
            lax.fori_loop(0, G, grp, 0)
            for s in range(NSLOT):
                pltpu.async_copy(tb_hbm.at[idx_v.at[s]], val_v_of[id(idx_v)].at[s], sem)

        val_v_of = {id(idx_a): val_a, id(idx_b): val_b}

        def drain(idx_v, sem):
            val_v = val_v_of[id(idx_v)]
            for s in range(NSLOT):
                pltpu.make_async_copy(tb_hbm.at[idx_v.at[s]],
                                      val_v.at[s], sem).wait()

        def math(t, idx_v, frac_v):
            val_v = val_v_of[id(idx_v)]

            def grp(g, carry):
                sl = pl.ds(g * 16, 16)
                fx = frac_v[0, sl]
                fy = frac_v[1, sl]
                fz = frac_v[2, sl]
                v = {cde: val_v[s, sl] for s, cde in enumerate(_SLOTS)}
                c00 = v[1, 1, 1] * (1.0 - fx) + v[2, 1, 1] * fx
                c01 = v[1, 1, 2] * (1.0 - fx) + v[2, 1, 2] * fx
                c10 = v[1, 2, 1] * (1.0 - fx) + v[2, 2, 1] * fx
                c11 = v[1, 2, 2] * (1.0 - fx) + v[2, 2, 2] * fx
                c0 = c00 * (1.0 - fy) + c10 * fy
                c1 = c01 * (1.0 - fy) + c11 * fy
                cval = c0 * (1.0 - fz) + c1 * fz
                mdx = mdy = mdz = None
                for (c, d, e) in _D2_CORNERS:
                    d2x = jnp.abs(v[c + 1, d, e] - 2.0 * v[c, d, e]
                                  + v[c - 1, d, e])
                    d2y = jnp.abs(v[c, d + 1, e] - 2.0 * v[c, d, e]
                                  + v[c, d - 1, e])
                    d2z = jnp.abs(v[c, d, e + 1] - 2.0 * v[c, d, e]
                                  + v[c, d, e - 1])
                    mdx = d2x if mdx is None else jnp.minimum(mdx, d2x)
                    mdy = d2y if mdy is None else jnp.minimum(mdy, d2y)
                    mdz = d2z if mdz is None else jnp.minimum(mdz, d2z)
                cval = (cval
                        - mdx * 0.5 * fx * (1.0 - fx)
                        - mdy * 0.5 * fy * (1.0 - fy)
                        - mdz * 0.5 * fz * (1.0 - fz))
                rv[pl.ds(t * C + g * 16, 16)] = cval
                return carry
            lax.fori_loop(0, G, grp, 0)

        fire(0, idx_a, frac_a, sem_a)

        def pipe(k, carry):
            te = 2 * k
            fire(te + 1, idx_b, frac_b, sem_b)
            drain(idx_a, sem_a)
            math(te, idx_a, frac_a)

            @pl.when(k < NCHUNK // 2 - 1)
            def _():
                fire(te + 2, idx_a, frac_a, sem_a)

            drain(idx_b, sem_b)
            math(te + 1, idx_b, frac_b)
            return carry

        lax.fori_loop(0, NCHUNK // 2, pipe, 0)
        pltpu.sync_copy(rv, out_hbm.at[pl.ds(tbase, PTS_PER_W)])

    return body


_SC_CALL = _make_sc_call()


def kernel(r, phi_r, trainables_m, trainables_p):
    xc = jnp.linspace(-1.0, 1.0, NX, dtype=jnp.float32)
    dx = xc[1] - xc[0]
    xg = jnp.concatenate([xc[:1] - dx, xc, xc[-1:] + dx])
    xg = jnp.pad(xg, (0, 384 - NX - 2))
    tb = jnp.concatenate([trainables_m.reshape(-1), trainables_p.reshape(-1)])
    cst = jnp.concatenate([jnp.full((16,), xg[0]), jnp.full((16,), dx)])
    rt = r.T
    return _SC_CALL(rt[0], rt[1], rt[2], phi_r, tb, xg, cst)


# speedup vs baseline: 44.6703x; 1.0018x over previous
import functools

import jax
import jax.numpy as jnp
from jax import lax
from jax.experimental import pallas as pl
from jax.experimental.pallas import tpu as pltpu
from jax.experimental.pallas import tpu_sc as plsc

NX = NY = NZ = 256
NXYZ = NX * NY * NZ
N_POINTS = 524288
NW = 32
PTS_PER_W = N_POINTS // NW
C = 128
G = C // 16
NCHUNK = PTS_PER_W // C
NSLOT = 32

_SLOTS = (
    [(c, d, e) for (c, d) in ((1, 1), (1, 2), (2, 1), (2, 2)) for e in range(4)]
    + [(c, d, e) for c in (0, 3) for d in (1, 2) for e in (1, 2)]
    + [(c, d, e) for c in (1, 2) for d in (0, 3) for e in (1, 2)]
)
assert len(_SLOTS) == NSLOT
_D2_CORNERS = ((1, 1, 1), (2, 1, 1), (1, 2, 1), (1, 1, 2),
               (2, 1, 2), (1, 2, 2), (2, 2, 1), (2, 2, 2))


def _make_sc_call():
    mesh = plsc.VectorSubcoreMesh(core_axis_name="c", subcore_axis_name="s")

    @functools.partial(
        pl.kernel,
        out_type=jax.ShapeDtypeStruct((N_POINTS,), jnp.float32),
        mesh=mesh,
        compiler_params=pltpu.CompilerParams(needs_layout_passes=False,
                                             use_tc_tiling_on_sc=False),
        scratch_types=[
            pltpu.VMEM((384,), jnp.float32),
            pltpu.VMEM((32,), jnp.float32),
            pltpu.VMEM((PTS_PER_W,), jnp.float32),
            pltpu.VMEM((PTS_PER_W,), jnp.float32),
            pltpu.VMEM((PTS_PER_W,), jnp.float32),
            pltpu.VMEM((PTS_PER_W,), jnp.float32),
            pltpu.VMEM((PTS_PER_W,), jnp.float32),
            pltpu.VMEM((NSLOT, C), jnp.int32),
            pltpu.VMEM((NSLOT, C), jnp.int32),
            pltpu.VMEM((NSLOT, C), jnp.float32),
            pltpu.VMEM((NSLOT, C), jnp.float32),
            pltpu.VMEM((4, C), jnp.float32),
            pltpu.VMEM((4, C), jnp.float32),
            pltpu.SemaphoreType.DMA,
            pltpu.SemaphoreType.DMA,
        ],
    )
    def body(xs_hbm, ys_hbm, zs_hbm, phi_hbm, tb_hbm, xg_hbm, cst_hbm,
             out_hbm, coord_v, cst_v, xv, yv, zv, pv, rv,
             idx_a, idx_b, val_a, val_b, frac_a, frac_b, sem_a, sem_b):
        wid = lax.axis_index("s") * 2 + lax.axis_index("c")
        tbase = wid * PTS_PER_W
        pltpu.sync_copy(xg_hbm, coord_v)
        pltpu.sync_copy(cst_hbm, cst_v)
        pltpu.sync_copy(xs_hbm.at[pl.ds(tbase, PTS_PER_W)], xv)
        pltpu.sync_copy(ys_hbm.at[pl.ds(tbase, PTS_PER_W)], yv)
        pltpu.sync_copy(zs_hbm.at[pl.ds(tbase, PTS_PER_W)], zv)
        pltpu.sync_copy(phi_hbm.at[pl.ds(tbase, PTS_PER_W)], pv)
        xg0 = cst_v[pl.ds(0, 16)]
        dxv = cst_v[pl.ds(16, 16)]

        def axis_calc(p):
            t = (p - xg0) / dxv
            i = t.astype(jnp.int32)
            i = jnp.clip(i, 2, 256)
            ci = plsc.load_gather(coord_v, [i])
            ci1 = plsc.load_gather(coord_v, [i + 1])
            fd = (p - ci) / (ci1 - ci)
            a = i - 1
            return fd, (a - 1, a, jnp.minimum(a + 1, NX - 1),
                        jnp.minimum(a + 2, NX - 1))

        def fire(t, idx_v, frac_v, sem):
            def grp(g, carry):
                sl = pl.ds(g * 16, 16)
                psl = pl.ds(t * C + g * 16, 16)
                fx, xi = axis_calc(xv[psl])
                fy, yi = axis_calc(yv[psl])
                fz, zi = axis_calc(zv[psl])
                off = jnp.where(pv[psl] >= 0.0, jnp.int32(NXYZ), jnp.int32(0))
                frac_v[0, sl] = fx
                frac_v[1, sl] = fy
                frac_v[2, sl] = fz
                cols = {}
                for (c, d, _e) in _SLOTS:
                    if (c, d) not in cols:
                        cols[(c, d)] = (xi[c] * NY + yi[d]) * NZ + off
                for s, (c, d, e) in enumerate(_SLOTS):
                    idx_v[s, sl] = cols[(c, d)] + zi[e]
                return carry
            lax.fori_loop(0, G, grp, 0)
            for s in range(NSLOT):
                pltpu.async_copy(tb_hbm.at[idx_v.at[s]], val_v_of[id(idx_v)].at[s], sem)

        val_v_of = {id(idx_a): val_a, id(idx_b): val_b}

        def drain(idx_v, sem):
            val_v = val_v_of[id(idx_v)]
            for s in range(NSLOT):
                pltpu.make_async_copy(tb_hbm.at[idx_v.at[s]],
                                      val_v.at[s], sem).wait()

        def math(t, idx_v, frac_v):
            val_v = val_v_of[id(idx_v)]

            def grp(g, carry):
                sl = pl.ds(g * 16, 16)
                fx = frac_v[0, sl]
                fy = frac_v[1, sl]
                fz = frac_v[2, sl]
                v = {cde: val_v[s, sl] for s, cde in enumerate(_SLOTS)}
                c00 = v[1, 1, 1] * (1.0 - fx) + v[2, 1, 1] * fx
                c01 = v[1, 1, 2] * (1.0 - fx) + v[2, 1, 2] * fx
                c10 = v[1, 2, 1] * (1.0 - fx) + v[2, 2, 1] * fx
                c11 = v[1, 2, 2] * (1.0 - fx) + v[2, 2, 2] * fx
                c0 = c00 * (1.0 - fy) + c10 * fy
                c1 = c01 * (1.0 - fy) + c11 * fy
                cval = c0 * (1.0 - fz) + c1 * fz
                mdx = mdy = mdz = None
                for (c, d, e) in _D2_CORNERS:
                    d2x = jnp.abs(v[c + 1, d, e] - 2.0 * v[c, d, e]
                                  + v[c - 1, d, e])
                    d2y = jnp.abs(v[c, d + 1, e] - 2.0 * v[c, d, e]
                                  + v[c, d - 1, e])
                    d2z = jnp.abs(v[c, d, e + 1] - 2.0 * v[c, d, e]
                                  + v[c, d, e - 1])
                    mdx = d2x if mdx is None else jnp.minimum(mdx, d2x)
                    mdy = d2y if mdy is None else jnp.minimum(mdy, d2y)
                    mdz = d2z if mdz is None else jnp.minimum(mdz, d2z)
                cval = (cval
                        - mdx * 0.5 * fx * (1.0 - fx)
                        - mdy * 0.5 * fy * (1.0 - fy)
                        - mdz * 0.5 * fz * (1.0 - fz))
                rv[pl.ds(t * C + g * 16, 16)] = cval
                return carry
            lax.fori_loop(0, G, grp, 0)

        fire(0, idx_a, frac_a, sem_a)

        def pipe(k, carry):
            te = 2 * k
            fire(te + 1, idx_b, frac_b, sem_b)
            drain(idx_a, sem_a)
            math(te, idx_a, frac_a)

            @pl.when(k < NCHUNK // 2 - 1)
            def _():
                fire(te + 2, idx_a, frac_a, sem_a)

            drain(idx_b, sem_b)
            math(te + 1, idx_b, frac_b)
            return carry

        lax.fori_loop(0, NCHUNK // 2, pipe, 0)
        pltpu.sync_copy(rv, out_hbm.at[pl.ds(tbase, PTS_PER_W)])

    return body


_SC_CALL = _make_sc_call()


def kernel(r, phi_r, trainables_m, trainables_p):
    xc = jnp.linspace(-1.0, 1.0, NX, dtype=jnp.float32)
    dx = xc[1] - xc[0]
    xg = jnp.concatenate([xc[:1] - dx, xc, xc[-1:] + dx])
    xg = jnp.pad(xg, (0, 384 - NX - 2))
    tb = jnp.concatenate([trainables_m.reshape(-1), trainables_p.reshape(-1)])
    cst = jnp.concatenate([jnp.full((16,), xg[0]), jnp.full((16,), dx)])
    rt = r.T
    return _SC_CALL(rt[0], rt[1], rt[2], phi_r, tb, xg, cst)
